# baseline (speedup 1.0000x reference)
import jax
import jax.numpy as jnp
from jax.experimental import pallas as pl
from jax.experimental.pallas import tpu as pltpu

_NS = 4
_SD = 128
_K = 16
_OFFSETS = (1, 16, 256, 4096)
_BETA = 0.25


def _vq_block_kernel(x_ref, w_ref, wn2_ref, wsq_ref, zq_ref, ids_ref, loss_ref):
    x = x_ref[...]
    bt = x.shape[0]
    slane = jax.lax.broadcasted_iota(jnp.int32, (_K, bt), 0)
    pow2 = jax.lax.bitcast_convert_type(
        (jnp.int32(127 + _K - 1)
         - jax.lax.broadcasted_iota(jnp.int32, (1, _K), 1)) << jnp.int32(23),
        jnp.float32)
    packed_row = jnp.zeros((1, bt), dtype=jnp.int32)
    loss = jnp.float32(0.0)
    for i in range(_NS):
        xi = x[:, i * _SD:(i + 1) * _SD]
        wi = w_ref[i]
        dot2 = jax.lax.dot_general(
            xi, wn2_ref[i], (((1,), (1,)), ((), ())),
            preferred_element_type=jnp.float32)
        flat_sq = jnp.sum(xi * xi, axis=1, keepdims=True)
        score = (flat_sq + wsq_ref[i][None, :]) + dot2
        score_t = jnp.transpose(score)
        smin = jnp.min(score_t, axis=0, keepdims=True)
        mask = (score_t == smin).astype(jnp.float32)
        mval = jax.lax.dot_general(
            pow2, mask, (((1,), (0,)), ((), ())),
            preferred_element_type=jnp.float32)
        ids_row = (jnp.int32(_K - 1 + 127)
                   - (jax.lax.bitcast_convert_type(mval, jnp.int32)
                      >> jnp.int32(23)))
        onehot_t = (slane == ids_row).astype(jnp.float32)
        zq = jax.lax.dot_general(
            onehot_t, wi, (((0,), (0,)), ((), ())),
            preferred_element_type=jnp.float32)
        zq_ref[:, i * _SD:(i + 1) * _SD] = zq
        loss = loss + jnp.sum(smin)
        packed_row = packed_row + ids_row * jnp.int32(_OFFSETS[i])
    step = pl.program_id(0)
    ids_ref[0, pl.ds(step * bt, bt)] = packed_row[0]

    @pl.when(step == 0)
    def _init():
        loss_ref[...] = jnp.zeros((1, 1), jnp.float32)

    loss_ref[...] += loss.reshape(1, 1)


def kernel(h, W):
    Bq, Nq, Dq = h.shape
    tokens = Bq * Nq
    bt = 4096
    grid = tokens // bt
    hf = h.reshape(tokens, Dq)
    W_sq = jnp.sum(W * W, axis=2)
    W_n2 = W * jnp.float32(-2.0)

    zq, ids3, loss = pl.pallas_call(
        _vq_block_kernel,
        grid=(grid,),
        in_specs=[
            pl.BlockSpec((bt, Dq), lambda i: (i, 0)),
            pl.BlockSpec((_NS, _K, _SD), lambda i: (0, 0, 0)),
            pl.BlockSpec((_NS, _K, _SD), lambda i: (0, 0, 0)),
            pl.BlockSpec((_NS, _K), lambda i: (0, 0)),
        ],
        out_specs=[
            pl.BlockSpec((bt, Dq), lambda i: (i, 0)),
            pl.BlockSpec((1, tokens), lambda i: (0, 0)),
            pl.BlockSpec((1, 1), lambda i: (0, 0)),
        ],
        out_shape=[
            jax.ShapeDtypeStruct((tokens, Dq), jnp.float32),
            jax.ShapeDtypeStruct((1, tokens), jnp.int32),
            jax.ShapeDtypeStruct((1, 1), jnp.float32),
        ],
        compiler_params=pltpu.CompilerParams(
            dimension_semantics=("arbitrary",)),
    )(hf, W, W_n2, W_sq)

    z_q = zq.reshape(Bq, Nq, Dq)
    ids_packed = ids3.reshape(Bq, Nq)
    n_elems = jnp.float32(tokens * _SD)
    vq_total = (loss[0, 0] * jnp.float32(1.0 + _BETA)) / n_elems
    return (z_q, ids_packed, vq_total)
